# Initial kernel scaffold; baseline (speedup 1.0000x reference)
import jax
import jax.numpy as jnp
from jax.experimental import pallas as pl


def kernel(x, edge_index, batch, W1, b1, W2, b2, Wc, bc):
    raise NotImplementedError("write your pallas kernel here")


# speedup vs baseline: 20.2136x; 20.2136x over previous
import functools

import jax
import jax.numpy as jnp
from jax import lax
from jax.experimental import pallas as pl
from jax.experimental.pallas import tpu as pltpu
from jax.experimental.pallas import tpu_sc as plsc

_NC = 2
_NS = 16
_NW = _NC * _NS
_K = 80
_G = 128


def _sc_mesh():
  return plsc.VectorSubcoreMesh(core_axis_name="c", subcore_axis_name="s",
                                num_cores=_NC, num_subcores=_NS)


def _deg_partials(col3, N):
  C = col3.shape[1]
  npad = ((N + 2047) // 2048) * 2048
  span = npad // _NS

  @functools.partial(
      pl.kernel,
      mesh=_sc_mesh(),
      out_type=jax.ShapeDtypeStruct((_NC, 1, npad), jnp.float32),
      scratch_types=[
          pltpu.VMEM((span,), jnp.float32),
          pltpu.VMEM((C, _K), jnp.int32),
          pltpu.VMEM((_K,), jnp.float32),
          pltpu.VMEM_SHARED((npad,), jnp.float32),
      ],
  )
  def deg_kernel(col_hbm, out_hbm, zbuf, colv, ones, acc):
    cid = lax.axis_index("c")
    sid = lax.axis_index("s")
    wid = cid * _NS + sid
    for k in range(span // 16):
      zbuf[pl.ds(k * 16, 16)] = jnp.zeros((16,), jnp.float32)
    for k in range(_K // 16):
      ones[pl.ds(k * 16, 16)] = jnp.ones((16,), jnp.float32)
    off = pl.multiple_of(sid * span, 128)
    pltpu.sync_copy(zbuf, acc.at[pl.ds(off, span)])
    pltpu.sync_copy(col_hbm.at[wid], colv)
    plsc.subcore_barrier()

    def body(j, carry):
      pltpu.sync_copy(ones, acc.at[colv.at[j]], add=True)
      return carry

    lax.fori_loop(0, C, body, 0)
    plsc.subcore_barrier()
    pltpu.sync_copy(acc.at[pl.ds(off, span)],
                    out_hbm.at[cid, 0, pl.ds(off, span)])

  return deg_kernel(col3)


def _mp_partials(y, row3, col3):
  N, H = y.shape
  C = row3.shape[1]
  span = ((N // _NS + 7) // 8) * 8
  last = N - span * (_NS - 1)

  @functools.partial(
      pl.kernel,
      mesh=_sc_mesh(),
      out_type=jax.ShapeDtypeStruct((_NC, N, H), jnp.float32),
      scratch_types=[
          pltpu.VMEM((C, _K), jnp.int32),
          pltpu.VMEM((C, _K), jnp.int32),
          pltpu.VMEM((_K, H), jnp.float32),
          pltpu.VMEM_SHARED((N, H), jnp.float32),
          pltpu.SemaphoreType.DMA,
      ],
  )
  def mp_kernel(y_hbm, row_hbm, col_hbm, out_hbm, rowv, colv, rbuf, acc, sem):
    cid = lax.axis_index("c")
    sid = lax.axis_index("s")
    wid = cid * _NS + sid
    off = pl.multiple_of(sid * span, 8)

    @pl.when(sid < _NS - 1)
    def _init_main():
      pltpu.sync_copy(y_hbm.at[pl.ds(off, span)], acc.at[pl.ds(off, span)])

    @pl.when(sid == _NS - 1)
    def _init_last():
      pltpu.sync_copy(y_hbm.at[pl.ds(span * (_NS - 1), last)],
                      acc.at[pl.ds(span * (_NS - 1), last)])

    pltpu.sync_copy(row_hbm.at[wid], rowv)
    pltpu.sync_copy(col_hbm.at[wid], colv)
    plsc.subcore_barrier()

    def body(j, carry):
      pltpu.async_copy(y_hbm.at[rowv.at[j]], rbuf, sem).wait()
      pltpu.sync_copy(rbuf, acc.at[colv.at[j]], add=True)
      return carry

    lax.fori_loop(0, C, body, 0)
    plsc.subcore_barrier()

    @pl.when(sid < _NS - 1)
    def _out_main():
      pltpu.sync_copy(acc.at[pl.ds(off, span)],
                      out_hbm.at[cid, pl.ds(off, span)])

    @pl.when(sid == _NS - 1)
    def _out_last():
      pltpu.sync_copy(acc.at[pl.ds(span * (_NS - 1), last)],
                      out_hbm.at[cid, pl.ds(span * (_NS - 1), last)])

  return mp_kernel(y, row3, col3)


def _tc_in(x, W, degs):
  N, D = x.shape
  H = W.shape[1]
  BN = 1000

  def body(xr, wr, dr, orf):
    dinv = lax.rsqrt(dr[0] + dr[1] + 1.0)
    orf[...] = jnp.dot(xr[...], wr[...],
                       preferred_element_type=jnp.float32) * dinv

  return pl.pallas_call(
      body,
      grid=(N // BN,),
      in_specs=[
          pl.BlockSpec((BN, D), lambda i: (i, 0)),
          pl.BlockSpec((D, H), lambda i: (0, 0)),
          pl.BlockSpec((2, BN, 1), lambda i: (0, i, 0)),
      ],
      out_specs=pl.BlockSpec((BN, H), lambda i: (i, 0)),
      out_shape=jax.ShapeDtypeStruct((N, H), jnp.float32),
  )(x, W, degs)


def _tc_mid(acc, y, degs, b, W):
  N, H = y.shape
  H2 = W.shape[1]
  BN = 1000

  def body(ar, yr, dr, br, wr, orf):
    dinv = lax.rsqrt(dr[0] + dr[1] + 1.0)
    h = jnp.maximum((ar[0] + ar[1] - yr[...]) * dinv + br[...], 0.0)
    orf[...] = jnp.dot(h, wr[...], preferred_element_type=jnp.float32) * dinv

  return pl.pallas_call(
      body,
      grid=(N // BN,),
      in_specs=[
          pl.BlockSpec((2, BN, H), lambda i: (0, i, 0)),
          pl.BlockSpec((BN, H), lambda i: (i, 0)),
          pl.BlockSpec((2, BN, 1), lambda i: (0, i, 0)),
          pl.BlockSpec((1, H), lambda i: (0, 0)),
          pl.BlockSpec((H, H2), lambda i: (0, 0)),
      ],
      out_specs=pl.BlockSpec((BN, H2), lambda i: (i, 0)),
      out_shape=jax.ShapeDtypeStruct((N, H2), jnp.float32),
  )(acc, y, degs, b, W)


def _tc_pool(acc, y, degs, b, batch3, Wc, bc):
  N, H = y.shape
  O = Wc.shape[1]
  BN = 1000

  def body(ar, yr, dr, br, batchr, wcr, bcr, orf, sums, counts):
    i = pl.program_id(0)

    @pl.when(i == 0)
    def _init():
      sums[...] = jnp.zeros_like(sums)
      counts[...] = jnp.zeros_like(counts)

    dinv = lax.rsqrt(dr[0] + dr[1] + 1.0)
    h = jnp.maximum((ar[0] + ar[1] - yr[...]) * dinv + br[...], 0.0)
    seg = batchr[0]
    gids = lax.broadcasted_iota(jnp.int32, (_G, 1), 0)
    m = jnp.where(seg == gids, 1.0, 0.0)
    sums[...] += jnp.dot(m, h, preferred_element_type=jnp.float32)
    counts[...] += jnp.sum(m, axis=1, keepdims=True)

    @pl.when(i == pl.num_programs(0) - 1)
    def _fin():
      hg = sums[...] / jnp.maximum(counts[...], 1.0)
      orf[...] = jnp.dot(hg, wcr[...],
                         preferred_element_type=jnp.float32) + bcr[...]

  return pl.pallas_call(
      body,
      grid=(N // BN,),
      in_specs=[
          pl.BlockSpec((2, BN, H), lambda i: (0, i, 0)),
          pl.BlockSpec((BN, H), lambda i: (i, 0)),
          pl.BlockSpec((2, BN, 1), lambda i: (0, i, 0)),
          pl.BlockSpec((1, H), lambda i: (0, 0)),
          pl.BlockSpec((1, 1, BN), lambda i: (i, 0, 0)),
          pl.BlockSpec((H, O), lambda i: (0, 0)),
          pl.BlockSpec((1, O), lambda i: (0, 0)),
      ],
      out_specs=pl.BlockSpec((_G, O), lambda i: (0, 0)),
      out_shape=jax.ShapeDtypeStruct((_G, O), jnp.float32),
      scratch_shapes=[
          pltpu.VMEM((_G, H), jnp.float32),
          pltpu.VMEM((_G, 1), jnp.float32),
      ],
  )(acc, y, degs, b, batch3, Wc, bc)


def kernel(x, edge_index, batch, W1, b1, W2, b2, Wc, bc):
  N, D = x.shape
  E = edge_index.shape[1]
  H = W1.shape[1]
  BN = 1000
  assert E % (_NW * _K) == 0 and N % _NS == 0 and N % BN == 0

  row3 = edge_index[0].reshape(_NW, E // (_NW * _K), _K)
  col3 = edge_index[1].reshape(_NW, E // (_NW * _K), _K)

  degp = _deg_partials(col3, N)
  degs = degp[:, 0, :N].reshape(2, N, 1)

  y1 = _tc_in(x, W1, degs)
  a1 = _mp_partials(y1, row3, col3)
  y2 = _tc_mid(a1, y1, degs, b1.reshape(1, H), W2)
  a2 = _mp_partials(y2, row3, col3)
  batch3 = batch.reshape(N // BN, 1, BN)
  return _tc_pool(a2, y2, degs, b2.reshape(1, H), batch3, Wc,
                  bc.reshape(1, -1))


# speedup vs baseline: 29.9298x; 1.4807x over previous
import functools

import jax
import jax.numpy as jnp
from jax import lax
from jax.experimental import pallas as pl
from jax.experimental.pallas import tpu as pltpu
from jax.experimental.pallas import tpu_sc as plsc

_NC = 2
_NS = 16
_NW = _NC * _NS
_K = 80
_G = 128


def _sc_mesh():
  return plsc.VectorSubcoreMesh(core_axis_name="c", subcore_axis_name="s",
                                num_cores=_NC, num_subcores=_NS)


def _deg_partials(col4, N):
  nseg, seg = col4.shape[1], col4.shape[2]
  npad = ((N + 2047) // 2048) * 2048
  span = npad // _NS

  @functools.partial(
      pl.kernel,
      mesh=_sc_mesh(),
      out_type=jax.ShapeDtypeStruct((_NC, 1, npad), jnp.float32),
      scratch_types=[
          pltpu.VMEM((span,), jnp.float32),
          pltpu.VMEM((seg, _K), jnp.int32),
          pltpu.VMEM((_K,), jnp.float32),
          pltpu.VMEM_SHARED((npad,), jnp.float32),
      ],
  )
  def deg_kernel(col_hbm, out_hbm, zbuf, colv, ones, acc):
    cid = lax.axis_index("c")
    sid = lax.axis_index("s")
    wid = cid * _NS + sid
    for k in range(span // 16):
      zbuf[pl.ds(k * 16, 16)] = jnp.zeros((16,), jnp.float32)
    for k in range(_K // 16):
      ones[pl.ds(k * 16, 16)] = jnp.ones((16,), jnp.float32)
    off = pl.multiple_of(sid * span, 128)
    pltpu.sync_copy(zbuf, acc.at[pl.ds(off, span)])
    plsc.subcore_barrier()

    for s in range(nseg):
      pltpu.sync_copy(col_hbm.at[wid, s], colv)

      def body(j, carry):
        pltpu.sync_copy(ones, acc.at[colv.at[j]], add=True)
        return carry

      lax.fori_loop(0, seg, body, 0)

    plsc.subcore_barrier()
    pltpu.sync_copy(acc.at[pl.ds(off, span)],
                    out_hbm.at[cid, 0, pl.ds(off, span)])

  return deg_kernel(col4)


def _mp_partials(y, row4, col4):
  N, H = y.shape
  nseg, seg = row4.shape[1], row4.shape[2]
  span = ((N // _NS + 7) // 8) * 8
  last = N - span * (_NS - 1)

  @functools.partial(
      pl.kernel,
      mesh=_sc_mesh(),
      out_type=jax.ShapeDtypeStruct((_NC, N, H), jnp.float32),
      scratch_types=[
          pltpu.VMEM((2, seg, _K), jnp.int32),
          pltpu.VMEM((2, seg, _K), jnp.int32),
          pltpu.VMEM((2, _K, H), jnp.float32),
          pltpu.VMEM_SHARED((N, H), jnp.float32),
          pltpu.SemaphoreType.DMA,
          pltpu.SemaphoreType.DMA,
          pltpu.SemaphoreType.DMA,
      ],
  )
  def mp_kernel(y_hbm, row_hbm, col_hbm, out_hbm, rowv, colv, rbuf, acc,
                sem0, sem1, isem):
    cid = lax.axis_index("c")
    sid = lax.axis_index("s")
    wid = cid * _NS + sid
    off = pl.multiple_of(sid * span, 8)

    @pl.when(sid < _NS - 1)
    def _init_main():
      pltpu.sync_copy(y_hbm.at[pl.ds(off, span)], acc.at[pl.ds(off, span)])

    @pl.when(sid == _NS - 1)
    def _init_last():
      pltpu.sync_copy(y_hbm.at[pl.ds(span * (_NS - 1), last)],
                      acc.at[pl.ds(span * (_NS - 1), last)])

    def _load_idx(s, ib):
      pltpu.async_copy(row_hbm.at[wid, s], rowv.at[ib], isem)
      pltpu.async_copy(col_hbm.at[wid, s], colv.at[ib], isem)

    def _wait_idx(s, ib):
      pltpu.make_async_copy(row_hbm.at[wid, s], rowv.at[ib], isem).wait()
      pltpu.make_async_copy(col_hbm.at[wid, s], colv.at[ib], isem).wait()

    _load_idx(0, 0)
    _wait_idx(0, 0)
    plsc.subcore_barrier()

    sems = (sem0, sem1)

    def _gather(ib, j, b):
      pltpu.async_copy(y_hbm.at[rowv.at[ib, j]], rbuf.at[b], sems[b])

    def _wait(ib, j, b):
      pltpu.make_async_copy(y_hbm.at[rowv.at[ib, j]], rbuf.at[b],
                            sems[b]).wait()

    def _scatter(ib, j, b):
      pltpu.sync_copy(rbuf.at[b], acc.at[colv.at[ib, j]], add=True)

    for s in range(nseg):
      ib = s % 2
      if s + 1 < nseg:
        _load_idx(s + 1, 1 - ib)
      _gather(ib, 0, 0)

      def body(p, carry, ib=ib):
        j = p * 2 + 1
        _gather(ib, j, 1)
        _wait(ib, j - 1, 0)
        _scatter(ib, j - 1, 0)
        _gather(ib, j + 1, 0)
        _wait(ib, j, 1)
        _scatter(ib, j, 1)
        return carry

      lax.fori_loop(0, (seg - 1) // 2, body, 0)
      _wait(ib, seg - 1, 0)
      _scatter(ib, seg - 1, 0)
      if s + 1 < nseg:
        _wait_idx(s + 1, 1 - ib)

    plsc.subcore_barrier()

    @pl.when(sid < _NS - 1)
    def _out_main():
      pltpu.sync_copy(acc.at[pl.ds(off, span)],
                      out_hbm.at[cid, pl.ds(off, span)])

    @pl.when(sid == _NS - 1)
    def _out_last():
      pltpu.sync_copy(acc.at[pl.ds(span * (_NS - 1), last)],
                      out_hbm.at[cid, pl.ds(span * (_NS - 1), last)])

  return mp_kernel(y, row4, col4)


def _tc_in(x, W, degs):
  N, D = x.shape
  H = W.shape[1]
  BN = 1000

  def body(xr, wr, dr, orf):
    dinv = lax.rsqrt(dr[0] + dr[1] + 1.0)
    orf[...] = jnp.dot(xr[...], wr[...],
                       preferred_element_type=jnp.float32) * dinv

  return pl.pallas_call(
      body,
      grid=(N // BN,),
      in_specs=[
          pl.BlockSpec((BN, D), lambda i: (i, 0)),
          pl.BlockSpec((D, H), lambda i: (0, 0)),
          pl.BlockSpec((2, BN, 1), lambda i: (0, i, 0)),
      ],
      out_specs=pl.BlockSpec((BN, H), lambda i: (i, 0)),
      out_shape=jax.ShapeDtypeStruct((N, H), jnp.float32),
  )(x, W, degs)


def _tc_mid(acc, y, degs, b, W):
  N, H = y.shape
  H2 = W.shape[1]
  BN = 1000

  def body(ar, yr, dr, br, wr, orf):
    dinv = lax.rsqrt(dr[0] + dr[1] + 1.0)
    h = jnp.maximum((ar[0] + ar[1] - yr[...]) * dinv + br[...], 0.0)
    orf[...] = jnp.dot(h, wr[...], preferred_element_type=jnp.float32) * dinv

  return pl.pallas_call(
      body,
      grid=(N // BN,),
      in_specs=[
          pl.BlockSpec((2, BN, H), lambda i: (0, i, 0)),
          pl.BlockSpec((BN, H), lambda i: (i, 0)),
          pl.BlockSpec((2, BN, 1), lambda i: (0, i, 0)),
          pl.BlockSpec((1, H), lambda i: (0, 0)),
          pl.BlockSpec((H, H2), lambda i: (0, 0)),
      ],
      out_specs=pl.BlockSpec((BN, H2), lambda i: (i, 0)),
      out_shape=jax.ShapeDtypeStruct((N, H2), jnp.float32),
  )(acc, y, degs, b, W)


def _tc_pool(acc, y, degs, b, batch3, Wc, bc):
  N, H = y.shape
  O = Wc.shape[1]
  BN = 1000

  def body(ar, yr, dr, br, batchr, wcr, bcr, orf, sums, counts):
    i = pl.program_id(0)

    @pl.when(i == 0)
    def _init():
      sums[...] = jnp.zeros_like(sums)
      counts[...] = jnp.zeros_like(counts)

    dinv = lax.rsqrt(dr[0] + dr[1] + 1.0)
    h = jnp.maximum((ar[0] + ar[1] - yr[...]) * dinv + br[...], 0.0)
    seg = batchr[0]
    gids = lax.broadcasted_iota(jnp.int32, (_G, 1), 0)
    m = jnp.where(seg == gids, 1.0, 0.0)
    sums[...] += jnp.dot(m, h, preferred_element_type=jnp.float32)
    counts[...] += jnp.sum(m, axis=1, keepdims=True)

    @pl.when(i == pl.num_programs(0) - 1)
    def _fin():
      hg = sums[...] / jnp.maximum(counts[...], 1.0)
      orf[...] = jnp.dot(hg, wcr[...],
                         preferred_element_type=jnp.float32) + bcr[...]

  return pl.pallas_call(
      body,
      grid=(N // BN,),
      in_specs=[
          pl.BlockSpec((2, BN, H), lambda i: (0, i, 0)),
          pl.BlockSpec((BN, H), lambda i: (i, 0)),
          pl.BlockSpec((2, BN, 1), lambda i: (0, i, 0)),
          pl.BlockSpec((1, H), lambda i: (0, 0)),
          pl.BlockSpec((1, 1, BN), lambda i: (i, 0, 0)),
          pl.BlockSpec((H, O), lambda i: (0, 0)),
          pl.BlockSpec((1, O), lambda i: (0, 0)),
      ],
      out_specs=pl.BlockSpec((_G, O), lambda i: (0, 0)),
      out_shape=jax.ShapeDtypeStruct((_G, O), jnp.float32),
      scratch_shapes=[
          pltpu.VMEM((_G, H), jnp.float32),
          pltpu.VMEM((_G, 1), jnp.float32),
      ],
  )(acc, y, degs, b, batch3, Wc, bc)


def kernel(x, edge_index, batch, W1, b1, W2, b2, Wc, bc):
  N, D = x.shape
  E = edge_index.shape[1]
  H = W1.shape[1]
  BN = 1000
  assert E % (_NW * _K) == 0 and N % _NS == 0 and N % BN == 0

  seg = 25
  nseg = E // (_NW * _K * seg)
  row4 = edge_index[0].reshape(_NW, nseg, seg, _K)
  col4 = edge_index[1].reshape(_NW, nseg, seg, _K)

  degp = _deg_partials(col4, N)
  degs = degp[:, 0, :N].reshape(2, N, 1)

  y1 = _tc_in(x, W1, degs)
  a1 = _mp_partials(y1, row4, col4)
  y2 = _tc_mid(a1, y1, degs, b1.reshape(1, H), W2)
  a2 = _mp_partials(y2, row4, col4)
  batch3 = batch.reshape(N // BN, 1, BN)
  return _tc_pool(a2, y2, degs, b2.reshape(1, H), batch3, Wc,
                  bc.reshape(1, -1))
